# baseline (speedup 1.0000x reference)
import functools

import jax
import jax.numpy as jnp
from jax import lax
from jax.experimental import pallas as pl
from jax.experimental.pallas import tpu as pltpu
from jax.experimental.pallas import tpu_sc as plsc

B = 1024
S = 200
D = 128
NC, NS, L = 2, 16, 16
NW = NC * NS
BPW = B // NW
NREG = D // L
TPAD = 208


def _renorm_body(w_ref, out_ref):
    w = w_ref[...]
    n = jnp.sqrt(jnp.sum(w * w, axis=1, keepdims=True))
    scale = jnp.where(n > 1.0, 1.0 / (n + 1e-7), 1.0)
    out_ref[...] = w * scale


def _renorm_tables(tables):
    return pl.pallas_call(
        _renorm_body,
        out_shape=jax.ShapeDtypeStruct(tables.shape, tables.dtype),
    )(tables)


NBUF = 3


def _sc_body(in_hbm, bt_hbm, lh_hbm, tab_hbm, cls_hbm, out_hbm,
             pos_v, cls_v, idx_v, brow_v, lrow_v,
             buf0, buf1, buf2, gsem, is0, is1, is2, os0, os1, os2):
    wid = lax.axis_index("s") * NC + lax.axis_index("c")
    base = wid * BPW
    bufs = [buf0, buf1, buf2]
    isems = [is0, is1, is2]
    osems = [os0, os1, os2]

    def fire_in(g, k):
        pltpu.async_copy(in_hbm.at[base + g], bufs[k].at[pl.ds(1, S - 1)],
                         isems[k])

    def wait_in(k):
        pltpu.make_async_copy(in_hbm.at[0], bufs[k].at[pl.ds(1, S - 1)],
                              isems[k]).wait()

    def fire_out(g, k):
        pltpu.async_copy(bufs[k], out_hbm.at[base + g], osems[k])

    def wait_out(k):
        pltpu.make_async_copy(bufs[k], out_hbm.at[0], osems[k]).wait()

    def compute(k, g):
        buf = bufs[k]
        bias = [brow_v[g, pl.ds(j * L, L)] + lrow_v[g, pl.ds(j * L, L)]
                for j in range(NREG)]
        for j in range(NREG):
            buf[0, pl.ds(j * L, L)] = (cls_v[pl.ds(j * L, L)]
                                       + pos_v[0, pl.ds(j * L, L)] + bias[j])

        @pl.loop(1, S)
        def _row(s):
            for j in range(NREG):
                buf[s, pl.ds(j * L, L)] = (buf[s, pl.ds(j * L, L)]
                                           + pos_v[s, pl.ds(j * L, L)]
                                           + bias[j])

    pltpu.sync_copy(tab_hbm.at[pl.ds(0, S)], pos_v)
    pltpu.sync_copy(cls_hbm, cls_v)

    pltpu.sync_copy(bt_hbm.at[pl.ds(base, BPW)], idx_v)
    for j in range(BPW // L):
        idx_v[pl.ds(j * L, L)] = idx_v[pl.ds(j * L, L)] + S
    pltpu.async_copy(tab_hbm.at[idx_v], brow_v, gsem).wait()

    pltpu.sync_copy(lh_hbm.at[pl.ds(base, BPW)], idx_v)
    for j in range(BPW // L):
        idx_v[pl.ds(j * L, L)] = idx_v[pl.ds(j * L, L)] + (S + 2)
    pltpu.async_copy(tab_hbm.at[idx_v], lrow_v, gsem).wait()

    fire_in(0, 0)
    wait_in(0); fire_in(1, 1); compute(0, 0); fire_out(0, 0)
    wait_in(1); fire_in(2, 2); compute(1, 1); fire_out(1, 1)
    wait_in(2); wait_out(0); fire_in(3, 0); compute(2, 2); fire_out(2, 2)

    @pl.loop(NBUF, BPW - 2, step=NBUF)
    def _chunk(i):
        for k in range(NBUF):
            g = i + k
            kn = (k + 1) % NBUF
            wait_in(k)
            wait_out(kn)
            fire_in(g + 1, kn)
            compute(k, g)
            fire_out(g, k)

    wait_in(0); wait_out(1); fire_in(BPW - 1, 1); compute(0, BPW - 2)
    fire_out(BPW - 2, 0)
    wait_in(1); wait_out(2); compute(1, BPW - 1); fire_out(BPW - 1, 1)
    wait_out(0)
    wait_out(1)


@functools.partial(jax.jit, static_argnums=())
def _run_sc(input_segment, brush_type, is_left_handed, tables_r, cls_token):
    mesh = plsc.VectorSubcoreMesh(core_axis_name="c", subcore_axis_name="s",
                                  num_cores=NC, num_subcores=NS)
    f = pl.kernel(
        _sc_body,
        out_type=jax.ShapeDtypeStruct((B, S, D), jnp.float32),
        mesh=mesh,
        scratch_types=[
            pltpu.VMEM((S, D), jnp.float32),
            pltpu.VMEM((D,), jnp.float32),
            pltpu.VMEM((BPW,), jnp.int32),
            pltpu.VMEM((BPW, D), jnp.float32),
            pltpu.VMEM((BPW, D), jnp.float32),
            pltpu.VMEM((S, D), jnp.float32),
            pltpu.VMEM((S, D), jnp.float32),
            pltpu.VMEM((S, D), jnp.float32),
            pltpu.SemaphoreType.DMA,
            pltpu.SemaphoreType.DMA,
            pltpu.SemaphoreType.DMA,
            pltpu.SemaphoreType.DMA,
            pltpu.SemaphoreType.DMA,
            pltpu.SemaphoreType.DMA,
            pltpu.SemaphoreType.DMA,
        ],
        compiler_params=pltpu.CompilerParams(use_tc_tiling_on_sc=True),
    )
    return f(input_segment, brush_type, is_left_handed, tables_r, cls_token)


def kernel(input_segment, brush_type, is_left_handed, pos_emb, brush_emb,
           left_emb, cls_token):
    tables = jnp.concatenate(
        [pos_emb, brush_emb, left_emb,
         jnp.zeros((TPAD - S - 4, D), jnp.float32)], axis=0)
    tables_r = _renorm_tables(tables)
    return _run_sc(input_segment, brush_type, is_left_handed, tables_r,
                   cls_token)


# speedup vs baseline: 1.0704x; 1.0704x over previous
import functools

import jax
import jax.numpy as jnp
from jax import lax
from jax.experimental import pallas as pl
from jax.experimental.pallas import tpu as pltpu
from jax.experimental.pallas import tpu_sc as plsc

B = 1024
S = 200
D = 128
NC, NS, L = 2, 16, 16
NW = NC * NS
BPW = B // NW
NREG = D // L
TPAD = 208
BB = 32


def _renorm_body(w_ref, out_ref):
    w = w_ref[...]
    n = jnp.sqrt(jnp.sum(w * w, axis=1, keepdims=True))
    scale = jnp.where(n > 1.0, 1.0 / (n + 1e-7), 1.0)
    out_ref[...] = w * scale


def _renorm_tables(tables):
    return pl.pallas_call(
        _renorm_body,
        out_shape=jax.ShapeDtypeStruct(tables.shape, tables.dtype),
    )(tables)


def _sc_bias_body(bt_hbm, lh_hbm, tab_hbm, bias_hbm, idx_v, brow_v, lrow_v,
                  gsem):
    wid = lax.axis_index("s") * NC + lax.axis_index("c")
    base = wid * BPW

    pltpu.sync_copy(bt_hbm.at[pl.ds(base, BPW)], idx_v)
    for j in range(BPW // L):
        idx_v[pl.ds(j * L, L)] = idx_v[pl.ds(j * L, L)] + S
    pltpu.async_copy(tab_hbm.at[idx_v], brow_v, gsem).wait()

    pltpu.sync_copy(lh_hbm.at[pl.ds(base, BPW)], idx_v)
    for j in range(BPW // L):
        idx_v[pl.ds(j * L, L)] = idx_v[pl.ds(j * L, L)] + (S + 2)
    pltpu.async_copy(tab_hbm.at[idx_v], lrow_v, gsem).wait()

    @pl.loop(0, BPW)
    def _row(i):
        for j in range(NREG):
            brow_v[i, pl.ds(j * L, L)] = (brow_v[i, pl.ds(j * L, L)]
                                          + lrow_v[i, pl.ds(j * L, L)])

    pltpu.sync_copy(brow_v, bias_hbm.at[pl.ds(base, BPW)])


def _sc_bias(brush_type, is_left_handed, tables_r):
    mesh = plsc.VectorSubcoreMesh(core_axis_name="c", subcore_axis_name="s",
                                  num_cores=NC, num_subcores=NS)
    f = pl.kernel(
        _sc_bias_body,
        out_type=jax.ShapeDtypeStruct((B, D), jnp.float32),
        mesh=mesh,
        scratch_types=[
            pltpu.VMEM((BPW,), jnp.int32),
            pltpu.VMEM((BPW, D), jnp.float32),
            pltpu.VMEM((BPW, D), jnp.float32),
            pltpu.SemaphoreType.DMA,
        ],
    )
    return f(brush_type, is_left_handed, tables_r)


def _dense_body(bias_ref, pos_ref, cls_ref, in_ref, out_ref):
    bias = bias_ref[...]
    pos = pos_ref[...]
    out_ref[:, 0, :] = cls_ref[...] + pos[0:1, :] + bias
    out_ref[:, 1:, :] = (in_ref[...] + pos[1:, :][None, :, :]
                         + bias[:, None, :])


def _dense(bias, pos_r, cls2d, input_segment):
    return pl.pallas_call(
        _dense_body,
        grid=(B // BB,),
        in_specs=[
            pl.BlockSpec((BB, D), lambda i: (i, 0)),
            pl.BlockSpec((S, D), lambda i: (0, 0)),
            pl.BlockSpec((1, D), lambda i: (0, 0)),
            pl.BlockSpec((BB, S - 1, D), lambda i: (i, 0, 0)),
        ],
        out_specs=pl.BlockSpec((BB, S, D), lambda i: (i, 0, 0)),
        out_shape=jax.ShapeDtypeStruct((B, S, D), jnp.float32),
    )(bias, pos_r, cls2d, input_segment)


def kernel(input_segment, brush_type, is_left_handed, pos_emb, brush_emb,
           left_emb, cls_token):
    tables = jnp.concatenate(
        [pos_emb, brush_emb, left_emb,
         jnp.zeros((TPAD - S - 4, D), jnp.float32)], axis=0)
    tables_r = _renorm_tables(tables)
    bias = _sc_bias(brush_type, is_left_handed, tables_r)
    pos_r = tables_r[:S]
    return _dense(bias, pos_r, cls_token[None, :], input_segment)
